# Initial kernel scaffold; baseline (speedup 1.0000x reference)
import jax
import jax.numpy as jnp
from jax.experimental import pallas as pl


def kernel(x, rel_lon, rel_lat, coords_source, coords_target):
    raise NotImplementedError("write your pallas kernel here")


# speedup vs baseline: 5.5655x; 5.5655x over previous
import functools

import jax
import jax.numpy as jnp
from jax import lax
from jax.experimental import pallas as pl
from jax.experimental.pallas import tpu as pltpu
from jax.experimental.pallas import tpu_sc as plsc

_NH = 10
_TBLK = 256
_SC_CORES = 2
_SC_SUBCORES = 16
_NW = _SC_CORES * _SC_SUBCORES


def _topk_body(lon_ref, lat_ref, dist_ref, ilon_ref, ilat_ref, idx_ref):
    lon = lon_ref[...]
    lat = lat_ref[...]
    d = jnp.sqrt(lon * lon + lat * lat)
    n = d.shape[1]
    col = lax.broadcasted_iota(jnp.int32, d.shape, 1)
    work = d
    for k in range(_NH):
        m = jnp.min(work, axis=1, keepdims=True)
        ismin = work == m
        amin = jnp.min(jnp.where(ismin, col, n), axis=1, keepdims=True)
        sel = col == amin
        dist_ref[:, k:k + 1] = m
        ilon_ref[:, k:k + 1] = jnp.sum(jnp.where(sel, lon, 0.0), axis=1,
                                       keepdims=True)
        ilat_ref[:, k:k + 1] = jnp.sum(jnp.where(sel, lat, 0.0), axis=1,
                                       keepdims=True)
        idx_ref[:, k:k + 1] = amin
        if k + 1 < _NH:
            work = jnp.where(sel, jnp.float32(jnp.inf), work)


def _topk_call(rel_lon, rel_lat, *, interpret=False):
    n_t, n_s = rel_lon.shape
    grid = (n_t // _TBLK,)
    in_spec = pl.BlockSpec((_TBLK, n_s), lambda i: (i, 0))
    out_spec = pl.BlockSpec((_TBLK, _NH), lambda i: (i, 0))
    return pl.pallas_call(
        _topk_body,
        grid=grid,
        in_specs=[in_spec, in_spec],
        out_specs=[out_spec] * 4,
        out_shape=[
            jax.ShapeDtypeStruct((n_t, _NH), jnp.float32),
            jax.ShapeDtypeStruct((n_t, _NH), jnp.float32),
            jax.ShapeDtypeStruct((n_t, _NH), jnp.float32),
            jax.ShapeDtypeStruct((n_t, _NH), jnp.int32),
        ],
        interpret=interpret,
    )(rel_lon, rel_lat)


def _idw_body(ct_ref, cs_ref, x_ref, out_ref):
    dx = ct_ref[:, 0:1] - cs_ref[0:1, :]
    dy = ct_ref[:, 1:2] - cs_ref[1:2, :]
    w = 1.0 / (jnp.sqrt(dx * dx + dy * dy) + 1e-8)
    wsum = jnp.sum(w, axis=1, keepdims=True)
    b = x_ref.shape[0]
    for i in range(b):
        acc = jnp.dot(w, x_ref[i], preferred_element_type=jnp.float32)
        out_ref[i] = acc / wsum


def _idw_call(coords_target, coords_source_t, x, *, interpret=False):
    n_t = coords_target.shape[0]
    b, n_s, e = x.shape
    grid = (n_t // _TBLK,)
    return pl.pallas_call(
        _idw_body,
        grid=grid,
        in_specs=[
            pl.BlockSpec((_TBLK, 2), lambda i: (i, 0)),
            pl.BlockSpec((2, n_s), lambda i: (0, 0)),
            pl.BlockSpec((b, n_s, e), lambda i: (0, 0, 0)),
        ],
        out_specs=pl.BlockSpec((b, _TBLK, e), lambda i: (0, i, 0)),
        out_shape=jax.ShapeDtypeStruct((b, n_t, e), jnp.float32),
        interpret=interpret,
    )(coords_target, coords_source_t, x)


def _make_sc_gather(n_rows, d, chunk):
    assert n_rows % (_NW * chunk) == 0
    rows_per_w = n_rows // _NW
    n_chunks = rows_per_w // chunk
    mesh = plsc.VectorSubcoreMesh(
        core_axis_name="c", subcore_axis_name="s",
        num_cores=_SC_CORES, num_subcores=_SC_SUBCORES)

    @functools.partial(
        pl.kernel,
        out_type=jax.ShapeDtypeStruct((n_rows, d), jnp.float32),
        mesh=mesh,
        scratch_types=[
            pltpu.VMEM((chunk,), jnp.int32),
            pltpu.VMEM((chunk, d), jnp.float32),
            pltpu.SemaphoreType.DMA,
        ],
    )
    def gather(x_hbm, idx_hbm, out_hbm, idx_v, rows_v, sem):
        wid = lax.axis_index("s") * _SC_CORES + lax.axis_index("c")
        base = wid * rows_per_w

        def one_chunk(i, carry):
            off = base + i * chunk
            pltpu.sync_copy(idx_hbm.at[pl.ds(off, chunk)], idx_v)
            pltpu.async_copy(x_hbm.at[idx_v], rows_v, sem).wait()
            pltpu.sync_copy(rows_v, out_hbm.at[pl.ds(off, chunk)])
            return carry

        lax.fori_loop(0, n_chunks, one_chunk, 0)

    return gather


def kernel(x, rel_lon, rel_lat, coords_source, coords_target):
    b, n_s, e = x.shape
    n_t = rel_lon.shape[0]

    dist, ilon, ilat, idx = _topk_call(rel_lon, rel_lat)
    x_inter = _idw_call(coords_target, coords_source.T, x)

    x_t = x.transpose(1, 0, 2).reshape(n_s, b * e)
    gather = _make_sc_gather(n_t * _NH, b * e, chunk=640)
    rows = gather(x_t, idx.reshape(-1))
    x_nearest = rows.reshape(n_t, _NH, b, e).transpose(2, 0, 1, 3)

    return (x_nearest, x_inter, dist, ilon, ilat)
